# baseline (speedup 1.0000x reference)
import functools

import jax
import jax.numpy as jnp
import numpy as np
from jax import lax
from jax.experimental import pallas as pl
from jax.experimental.pallas import tpu as pltpu
from jax.experimental.pallas import tpu_sc as plsc

_B = 64
_V = 100000
_K = 64
_L = 16
_NCHUNK = 208
_CH = 512
_VPAD = _NCHUNK * _CH
_NG = _NCHUNK // _L
_WBLK = 2048
_NB = (_V + _WBLK - 1) // _WBLK
_BIG = 1 << 30


@functools.cache
def _g64_host():
    try:
        with jax.ensure_compile_time_eval():
            g = jax.random.gumbel(jax.random.key(42), (_B, _V),
                                  jnp.float32)[:, :_K]
            return np.asarray(g)
    except Exception:
        return None


def _g64():
    g = _g64_host()
    if g is not None:
        return jnp.asarray(g)
    return jax.random.gumbel(jax.random.key(42), (_B, _V), jnp.float32)[:, :_K]


def _sc_top64_body(logits_hbm, vals_hbm, idx_hbm, xbuf, cmax, vals_v, idx_v):
    wid = lax.axis_index("s") * 2 + lax.axis_index("c")
    iota = lax.iota(jnp.int32, _L)
    ninf = jnp.full((_L,), -jnp.inf, jnp.float32)
    bigv = jnp.full((_L,), _BIG, jnp.int32)
    lane0 = iota == 0

    def _tree_max(accs):
        return jnp.maximum(jnp.maximum(accs[0], accs[1]),
                           jnp.maximum(accs[2], accs[3]))

    def _chunk_scan(base):
        accs = [ninf, ninf, ninf, ninf]
        for j in range(_CH // _L):
            accs[j % 4] = jnp.maximum(accs[j % 4],
                                      xbuf[pl.ds(base + j * _L, _L)])
        return _tree_max(accs)

    for rr in range(2):
        r = wid * 2 + rr
        pltpu.sync_copy(logits_hbm.at[r], xbuf.at[pl.ds(0, _V)])

        def fill(t, _):
            xbuf[pl.ds(_V + t * _L, _L)] = ninf
            return 0
        lax.fori_loop(0, (_VPAD - _V) // _L, fill, 0)

        def cmax_body(ci, carry):
            smv, cmv = carry
            cm = jnp.max(_chunk_scan(ci * _CH), axis=0)
            u = jnp.bitwise_and(ci, _L - 1)
            cmv = jnp.where(iota == u, cm, cmv)
            done = u == _L - 1

            @pl.when(done)
            def _():
                cmax[pl.ds(ci - (_L - 1), _L)] = cmv
            smv = jnp.where(jnp.logical_and(done, iota == ci // _L),
                            jnp.max(cmv, axis=0), smv)
            cmv = jnp.where(done, ninf, cmv)
            return smv, cmv
        smv0, _unused = lax.fori_loop(0, _NCHUNK, cmax_body, (ninf, ninf))

        def extract(k, smv):
            gm = jnp.max(smv, axis=0)
            g = jnp.min(jnp.where(smv == gm, iota, bigv), axis=0)
            cv = cmax[pl.ds(g * _L, _L)]
            ci = jnp.min(jnp.where(cv == gm, g * _L + iota, bigv), axis=0)
            base = ci * _CH
            posv = bigv
            for j2 in range(_CH // _L):
                v = xbuf[pl.ds(base + j2 * _L, _L)]
                posv = jnp.minimum(posv,
                                   jnp.where(v == gm, j2 * _L + iota, bigv))
            gi = base + jnp.min(posv, axis=0)

            kf = jnp.full((_L,), k, jnp.int32)
            plsc.store_scatter(vals_v, [kf], jnp.full((_L,), gm, jnp.float32),
                               mask=lane0)
            plsc.store_scatter(idx_v, [kf], jnp.full((_L,), gi, jnp.int32),
                               mask=lane0)
            plsc.store_scatter(xbuf, [jnp.full((_L,), gi, jnp.int32)], ninf,
                               mask=lane0)
            nm = jnp.max(_chunk_scan(base), axis=0)
            plsc.store_scatter(cmax, [jnp.full((_L,), ci, jnp.int32)],
                               jnp.full((_L,), nm, jnp.float32), mask=lane0)
            cv2 = jnp.where(g * _L + iota == ci, nm, cv)
            return jnp.where(iota == g, jnp.max(cv2, axis=0), smv)
        lax.fori_loop(0, _K, extract, smv0)

        pltpu.sync_copy(vals_v, vals_hbm.at[pl.ds(r * _K, _K)])
        pltpu.sync_copy(idx_v, idx_hbm.at[pl.ds(r * _K, _K)])


def _sc_top64(logits):
    mesh = plsc.VectorSubcoreMesh(core_axis_name="c", subcore_axis_name="s")
    return pl.kernel(
        _sc_top64_body,
        mesh=mesh,
        out_type=(jax.ShapeDtypeStruct((_B * _K,), jnp.float32),
                  jax.ShapeDtypeStruct((_B * _K,), jnp.int32)),
        scratch_types=[pltpu.VMEM((_VPAD,), jnp.float32),
                       pltpu.VMEM((_NCHUNK,), jnp.float32),
                       pltpu.VMEM((_K,), jnp.float32),
                       pltpu.VMEM((_K,), jnp.int32)],
        compiler_params=pltpu.CompilerParams(needs_layout_passes=False,
                                             use_tc_tiling_on_sc=False),
    )(logits)


def _dense_body(x_ref, t_ref, o_ref, m_ref, s_ref):
    p = pl.program_id(0)
    i = pl.program_id(1)
    x = x_ref[...] * (1.0 / t_ref[...])

    @pl.when(p == 0)
    def _():
        col = lax.broadcasted_iota(jnp.int32, x.shape, 1) + i * _WBLK
        xm = jnp.where(col < _V, x, -jnp.inf)
        bm = jnp.max(xm, axis=1, keepdims=True)

        @pl.when(i == 0)
        def _():
            m_ref[...] = bm
            s_ref[...] = jnp.sum(jnp.exp(xm - bm), axis=1, keepdims=True)

        @pl.when(i > 0)
        def _():
            m0 = m_ref[...]
            nm = jnp.maximum(m0, bm)
            s_ref[...] = (s_ref[...] * jnp.exp(m0 - nm)
                          + jnp.sum(jnp.exp(xm - nm), axis=1, keepdims=True))
            m_ref[...] = nm

    @pl.when(p == 1)
    def _():
        o_ref[...] = x - (m_ref[...] + jnp.log(s_ref[...]))


def _sample_body(v_ref, i_ref, t_ref, m_ref, s_ref, tk_ref, tp_ref, mp_ref,
                 g_ref, o_ref):
    v = v_ref[...] / t_ref[...]
    p = jnp.exp(v - m_ref[...]) / s_ref[...]
    rank = lax.broadcasted_iota(jnp.int32, p.shape, 1)
    tri = (lax.broadcasted_iota(jnp.int32, (_K, _K), 0)
           <= lax.broadcasted_iota(jnp.int32, (_K, _K), 1)).astype(jnp.float32)
    cs = jnp.dot(p, tri, preferred_element_type=jnp.float32)
    pk = jnp.where(rank >= tk_ref[...], 0.0, p)
    pk = jnp.where(cs - p > tp_ref[...], 0.0, pk)
    pk = jnp.where(pk < pk[:, 0:1] * mp_ref[...], 0.0, pk)
    val = jnp.log(jnp.maximum(pk, 1e-30)) + g_ref[...]
    gm = jnp.max(val, axis=1, keepdims=True)
    pos = jnp.min(jnp.where(val == gm, rank, jnp.int32(_BIG)), axis=1,
                  keepdims=True)
    o_ref[...] = jnp.sum(jnp.where(rank == pos, i_ref[...], 0), axis=1,
                         keepdims=True)


def _tc_dense(logits, temperatures):
    return pl.pallas_call(
        _dense_body,
        grid=(2, _NB),
        in_specs=[pl.BlockSpec((_B, _WBLK), lambda p, i: (0, i)),
                  pl.BlockSpec((_B, 1), lambda p, i: (0, 0))],
        out_specs=[pl.BlockSpec((_B, _WBLK),
                                lambda p, i: (0, jnp.where(p == 1, i, 0))),
                   pl.BlockSpec((_B, 1), lambda p, i: (0, 0)),
                   pl.BlockSpec((_B, 1), lambda p, i: (0, 0))],
        out_shape=[jax.ShapeDtypeStruct((_B, _V), jnp.float32),
                   jax.ShapeDtypeStruct((_B, 1), jnp.float32),
                   jax.ShapeDtypeStruct((_B, 1), jnp.float32)],
    )(logits, temperatures)


def _tc_sample(vals, idx, temperatures, m, s, top_ks, top_ps, min_ps, g64):
    return pl.pallas_call(
        _sample_body,
        out_shape=jax.ShapeDtypeStruct((_B, 1), jnp.int32),
    )(vals, idx, temperatures, m, s, top_ks, top_ps, min_ps, g64)


def kernel(logits, temperatures, top_ks, top_ps, min_ps):
    vals, idx = _sc_top64(logits)
    vals = vals.reshape(_B, _K)
    idx = idx.reshape(_B, _K)
    logprobs, m, s = _tc_dense(logits, temperatures)
    ids = _tc_sample(vals, idx, temperatures, m, s,
                     top_ks.reshape(_B, 1), top_ps.reshape(_B, 1),
                     min_ps.reshape(_B, 1), _g64())
    return ids.reshape(_B), logprobs


# speedup vs baseline: 1.2335x; 1.2335x over previous
import functools

import jax
import jax.numpy as jnp
import numpy as np
from jax import lax
from jax.experimental import pallas as pl
from jax.experimental.pallas import tpu as pltpu
from jax.experimental.pallas import tpu_sc as plsc

_B = 64
_V = 100000
_K = 64
_L = 16
_NCHUNK = 208
_CH = 512
_VPAD = _NCHUNK * _CH
_NG = _NCHUNK // _L
_WBLK = 4096
_NB = (_V + _WBLK - 1) // _WBLK
_BIG = 1 << 30


@functools.cache
def _g64_host():
    try:
        with jax.ensure_compile_time_eval():
            g = jax.random.gumbel(jax.random.key(42), (_B, _V),
                                  jnp.float32)[:, :_K]
            return np.asarray(g)
    except Exception:
        return None


def _g64():
    g = _g64_host()
    if g is not None:
        return jnp.asarray(g)
    return jax.random.gumbel(jax.random.key(42), (_B, _V), jnp.float32)[:, :_K]


def _sc_top64_body(logits_hbm, vals_hbm, idx_hbm, xbuf, cmax, vals_v, idx_v):
    wid = lax.axis_index("s") * 2 + lax.axis_index("c")
    iota = lax.iota(jnp.int32, _L)
    ninf = jnp.full((_L,), -jnp.inf, jnp.float32)
    bigv = jnp.full((_L,), _BIG, jnp.int32)
    lane0 = iota == 0

    def _tree_max(accs):
        return jnp.maximum(jnp.maximum(accs[0], accs[1]),
                           jnp.maximum(accs[2], accs[3]))

    def _chunk_scan(base):
        accs = [ninf, ninf, ninf, ninf]
        for j in range(_CH // _L):
            accs[j % 4] = jnp.maximum(accs[j % 4],
                                      xbuf[pl.ds(base + j * _L, _L)])
        return _tree_max(accs)

    for rr in range(2):
        r = wid * 2 + rr
        pltpu.sync_copy(logits_hbm.at[r], xbuf.at[pl.ds(0, _V)])

        def fill(t, _):
            xbuf[pl.ds(_V + t * _L, _L)] = ninf
            return 0
        lax.fori_loop(0, (_VPAD - _V) // _L, fill, 0)

        def cmax_body(ci, carry):
            smv, cmv = carry
            cm = jnp.max(_chunk_scan(ci * _CH), axis=0)
            u = jnp.bitwise_and(ci, _L - 1)
            cmv = jnp.where(iota == u, cm, cmv)
            done = u == _L - 1

            @pl.when(done)
            def _():
                cmax[pl.ds(ci - (_L - 1), _L)] = cmv
            smv = jnp.where(jnp.logical_and(done, iota == ci // _L),
                            jnp.max(cmv, axis=0), smv)
            cmv = jnp.where(done, ninf, cmv)
            return smv, cmv
        smv0, _unused = lax.fori_loop(0, _NCHUNK, cmax_body, (ninf, ninf))

        def extract(k, smv):
            gm = jnp.max(smv, axis=0)
            g = jnp.min(jnp.where(smv == gm, iota, bigv), axis=0)
            cv = cmax[pl.ds(g * _L, _L)]
            ci = jnp.min(jnp.where(cv == gm, g * _L + iota, bigv), axis=0)
            base = ci * _CH
            posv = bigv
            for j2 in range(_CH // _L):
                v = xbuf[pl.ds(base + j2 * _L, _L)]
                posv = jnp.minimum(posv,
                                   jnp.where(v == gm, j2 * _L + iota, bigv))
            gi = base + jnp.min(posv, axis=0)

            kf = jnp.full((_L,), k, jnp.int32)
            plsc.store_scatter(vals_v, [kf], jnp.full((_L,), gm, jnp.float32),
                               mask=lane0)
            plsc.store_scatter(idx_v, [kf], jnp.full((_L,), gi, jnp.int32),
                               mask=lane0)
            plsc.store_scatter(xbuf, [jnp.full((_L,), gi, jnp.int32)], ninf,
                               mask=lane0)
            nm = jnp.max(_chunk_scan(base), axis=0)
            plsc.store_scatter(cmax, [jnp.full((_L,), ci, jnp.int32)],
                               jnp.full((_L,), nm, jnp.float32), mask=lane0)
            cv2 = jnp.where(g * _L + iota == ci, nm, cv)
            return jnp.where(iota == g, jnp.max(cv2, axis=0), smv)
        lax.fori_loop(0, _K, extract, smv0)

        pltpu.sync_copy(vals_v, vals_hbm.at[pl.ds(r * _K, _K)])
        pltpu.sync_copy(idx_v, idx_hbm.at[pl.ds(r * _K, _K)])


def _sc_top64(logits):
    mesh = plsc.VectorSubcoreMesh(core_axis_name="c", subcore_axis_name="s")
    return pl.kernel(
        _sc_top64_body,
        mesh=mesh,
        out_type=(jax.ShapeDtypeStruct((_B * _K,), jnp.float32),
                  jax.ShapeDtypeStruct((_B * _K,), jnp.int32)),
        scratch_types=[pltpu.VMEM((_VPAD,), jnp.float32),
                       pltpu.VMEM((_NCHUNK,), jnp.float32),
                       pltpu.VMEM((_K,), jnp.float32),
                       pltpu.VMEM((_K,), jnp.int32)],
        compiler_params=pltpu.CompilerParams(needs_layout_passes=False,
                                             use_tc_tiling_on_sc=False),
    )(logits)


def _dense_body(x_ref, t_ref, o_ref, m_ref, s_ref):
    p = pl.program_id(0)
    i = pl.program_id(1)
    x = x_ref[...] * (1.0 / t_ref[...])

    @pl.when(p == 0)
    def _():
        col = lax.broadcasted_iota(jnp.int32, x.shape, 1) + i * _WBLK
        xm = jnp.where(col < _V, x, -jnp.inf)
        bm = jnp.max(xm, axis=1, keepdims=True)

        @pl.when(i == 0)
        def _():
            m_ref[...] = bm
            s_ref[...] = jnp.sum(jnp.exp(xm - bm), axis=1, keepdims=True)

        @pl.when(i > 0)
        def _():
            m0 = m_ref[...]
            nm = jnp.maximum(m0, bm)
            s_ref[...] = (s_ref[...] * jnp.exp(m0 - nm)
                          + jnp.sum(jnp.exp(xm - nm), axis=1, keepdims=True))
            m_ref[...] = nm

    @pl.when(p == 1)
    def _():
        o_ref[...] = x - (m_ref[...] + jnp.log(s_ref[...]))


def _sample_body(v_ref, i_ref, t_ref, m_ref, s_ref, tk_ref, tp_ref, mp_ref,
                 g_ref, o_ref):
    v = v_ref[...] / t_ref[...]
    p = jnp.exp(v - m_ref[...]) / s_ref[...]
    rank = lax.broadcasted_iota(jnp.int32, p.shape, 1)
    tri = (lax.broadcasted_iota(jnp.int32, (_K, _K), 0)
           <= lax.broadcasted_iota(jnp.int32, (_K, _K), 1)).astype(jnp.float32)
    cs = jnp.dot(p, tri, preferred_element_type=jnp.float32)
    pk = jnp.where(rank >= tk_ref[...], 0.0, p)
    pk = jnp.where(cs - p > tp_ref[...], 0.0, pk)
    pk = jnp.where(pk < pk[:, 0:1] * mp_ref[...], 0.0, pk)
    val = jnp.log(jnp.maximum(pk, 1e-30)) + g_ref[...]
    gm = jnp.max(val, axis=1, keepdims=True)
    pos = jnp.min(jnp.where(val == gm, rank, jnp.int32(_BIG)), axis=1,
                  keepdims=True)
    o_ref[...] = jnp.sum(jnp.where(rank == pos, i_ref[...], 0), axis=1,
                         keepdims=True)


def _tc_dense(logits, temperatures):
    return pl.pallas_call(
        _dense_body,
        grid=(2, _NB),
        in_specs=[pl.BlockSpec((_B, _WBLK), lambda p, i: (0, i)),
                  pl.BlockSpec((_B, 1), lambda p, i: (0, 0))],
        out_specs=[pl.BlockSpec((_B, _WBLK),
                                lambda p, i: (0, jnp.where(p == 1, i, 0))),
                   pl.BlockSpec((_B, 1), lambda p, i: (0, 0)),
                   pl.BlockSpec((_B, 1), lambda p, i: (0, 0))],
        out_shape=[jax.ShapeDtypeStruct((_B, _V), jnp.float32),
                   jax.ShapeDtypeStruct((_B, 1), jnp.float32),
                   jax.ShapeDtypeStruct((_B, 1), jnp.float32)],
    )(logits, temperatures)


def _tc_sample(vals, idx, temperatures, m, s, top_ks, top_ps, min_ps, g64):
    return pl.pallas_call(
        _sample_body,
        out_shape=jax.ShapeDtypeStruct((_B, 1), jnp.int32),
    )(vals, idx, temperatures, m, s, top_ks, top_ps, min_ps, g64)


def kernel(logits, temperatures, top_ks, top_ps, min_ps):
    vals, idx = _sc_top64(logits)
    vals = vals.reshape(_B, _K)
    idx = idx.reshape(_B, _K)
    logprobs, m, s = _tc_dense(logits, temperatures)
    ids = _tc_sample(vals, idx, temperatures, m, s,
                     top_ks.reshape(_B, 1), top_ps.reshape(_B, 1),
                     min_ps.reshape(_B, 1), _g64())
    return ids.reshape(_B), logprobs


# speedup vs baseline: 1.3199x; 1.0700x over previous
import functools

import jax
import jax.numpy as jnp
import numpy as np
from jax import lax
from jax.experimental import pallas as pl
from jax.experimental.pallas import tpu as pltpu
from jax.experimental.pallas import tpu_sc as plsc

_B = 64
_V = 100000
_K = 64
_L = 16
_NCHUNK = 208
_CH = 512
_VPAD = _NCHUNK * _CH
_NG = _NCHUNK // _L
_WBLK = 8192
_NB = (_V + _WBLK - 1) // _WBLK
_BIG = 1 << 30


@functools.cache
def _g64_host():
    try:
        with jax.ensure_compile_time_eval():
            g = jax.random.gumbel(jax.random.key(42), (_B, _V),
                                  jnp.float32)[:, :_K]
            return np.asarray(g)
    except Exception:
        return None


def _g64():
    g = _g64_host()
    if g is not None:
        return jnp.asarray(g)
    return jax.random.gumbel(jax.random.key(42), (_B, _V), jnp.float32)[:, :_K]


def _sc_top64_body(logits_hbm, vals_hbm, idx_hbm, xbuf, cmax, vals_v, idx_v):
    wid = lax.axis_index("s") * 2 + lax.axis_index("c")
    iota = lax.iota(jnp.int32, _L)
    ninf = jnp.full((_L,), -jnp.inf, jnp.float32)
    bigv = jnp.full((_L,), _BIG, jnp.int32)
    lane0 = iota == 0

    def _tree_max(accs):
        return jnp.maximum(jnp.maximum(accs[0], accs[1]),
                           jnp.maximum(accs[2], accs[3]))

    def _chunk_scan(base):
        accs = [ninf, ninf, ninf, ninf]
        for j in range(_CH // _L):
            accs[j % 4] = jnp.maximum(accs[j % 4],
                                      xbuf[pl.ds(base + j * _L, _L)])
        return _tree_max(accs)

    for rr in range(2):
        r = wid * 2 + rr
        pltpu.sync_copy(logits_hbm.at[r], xbuf.at[pl.ds(0, _V)])

        def fill(t, _):
            xbuf[pl.ds(_V + t * _L, _L)] = ninf
            return 0
        lax.fori_loop(0, (_VPAD - _V) // _L, fill, 0)

        def cmax_body(ci, carry):
            smv, cmv = carry
            cm = jnp.max(_chunk_scan(ci * _CH), axis=0)
            u = jnp.bitwise_and(ci, _L - 1)
            cmv = jnp.where(iota == u, cm, cmv)
            done = u == _L - 1

            @pl.when(done)
            def _():
                cmax[pl.ds(ci - (_L - 1), _L)] = cmv
            smv = jnp.where(jnp.logical_and(done, iota == ci // _L),
                            jnp.max(cmv, axis=0), smv)
            cmv = jnp.where(done, ninf, cmv)
            return smv, cmv
        smv0, _unused = lax.fori_loop(0, _NCHUNK, cmax_body, (ninf, ninf))

        def extract(k, smv):
            gm = jnp.max(smv, axis=0)
            g = jnp.min(jnp.where(smv == gm, iota, bigv), axis=0)
            cv = cmax[pl.ds(g * _L, _L)]
            ci = jnp.min(jnp.where(cv == gm, g * _L + iota, bigv), axis=0)
            base = ci * _CH
            posv = bigv
            for j2 in range(_CH // _L):
                v = xbuf[pl.ds(base + j2 * _L, _L)]
                posv = jnp.minimum(posv,
                                   jnp.where(v == gm, j2 * _L + iota, bigv))
            gi = base + jnp.min(posv, axis=0)

            kf = jnp.full((_L,), k, jnp.int32)
            plsc.store_scatter(vals_v, [kf], jnp.full((_L,), gm, jnp.float32),
                               mask=lane0)
            plsc.store_scatter(idx_v, [kf], jnp.full((_L,), gi, jnp.int32),
                               mask=lane0)
            plsc.store_scatter(xbuf, [jnp.full((_L,), gi, jnp.int32)], ninf,
                               mask=lane0)
            nm = jnp.max(_chunk_scan(base), axis=0)
            plsc.store_scatter(cmax, [jnp.full((_L,), ci, jnp.int32)],
                               jnp.full((_L,), nm, jnp.float32), mask=lane0)
            cv2 = jnp.where(g * _L + iota == ci, nm, cv)
            return jnp.where(iota == g, jnp.max(cv2, axis=0), smv)
        lax.fori_loop(0, _K, extract, smv0)

        pltpu.sync_copy(vals_v, vals_hbm.at[pl.ds(r * _K, _K)])
        pltpu.sync_copy(idx_v, idx_hbm.at[pl.ds(r * _K, _K)])


def _sc_top64(logits):
    mesh = plsc.VectorSubcoreMesh(core_axis_name="c", subcore_axis_name="s")
    return pl.kernel(
        _sc_top64_body,
        mesh=mesh,
        out_type=(jax.ShapeDtypeStruct((_B * _K,), jnp.float32),
                  jax.ShapeDtypeStruct((_B * _K,), jnp.int32)),
        scratch_types=[pltpu.VMEM((_VPAD,), jnp.float32),
                       pltpu.VMEM((_NCHUNK,), jnp.float32),
                       pltpu.VMEM((_K,), jnp.float32),
                       pltpu.VMEM((_K,), jnp.int32)],
        compiler_params=pltpu.CompilerParams(needs_layout_passes=False,
                                             use_tc_tiling_on_sc=False),
    )(logits)


def _dense_body(x_ref, t_ref, o_ref, m_ref, s_ref):
    p = pl.program_id(0)
    i = pl.program_id(1)
    x = x_ref[...] * (1.0 / t_ref[...])

    @pl.when(p == 0)
    def _():
        col = lax.broadcasted_iota(jnp.int32, x.shape, 1) + i * _WBLK
        xm = jnp.where(col < _V, x, -jnp.inf)
        bm = jnp.max(xm, axis=1, keepdims=True)

        @pl.when(i == 0)
        def _():
            m_ref[...] = bm
            s_ref[...] = jnp.sum(jnp.exp(xm - bm), axis=1, keepdims=True)

        @pl.when(i > 0)
        def _():
            m0 = m_ref[...]
            nm = jnp.maximum(m0, bm)
            s_ref[...] = (s_ref[...] * jnp.exp(m0 - nm)
                          + jnp.sum(jnp.exp(xm - nm), axis=1, keepdims=True))
            m_ref[...] = nm

    @pl.when(p == 1)
    def _():
        o_ref[...] = x - (m_ref[...] + jnp.log(s_ref[...]))


def _sample_body(v_ref, i_ref, t_ref, m_ref, s_ref, tk_ref, tp_ref, mp_ref,
                 g_ref, o_ref):
    v = v_ref[...] / t_ref[...]
    p = jnp.exp(v - m_ref[...]) / s_ref[...]
    rank = lax.broadcasted_iota(jnp.int32, p.shape, 1)
    tri = (lax.broadcasted_iota(jnp.int32, (_K, _K), 0)
           <= lax.broadcasted_iota(jnp.int32, (_K, _K), 1)).astype(jnp.float32)
    cs = jnp.dot(p, tri, preferred_element_type=jnp.float32)
    pk = jnp.where(rank >= tk_ref[...], 0.0, p)
    pk = jnp.where(cs - p > tp_ref[...], 0.0, pk)
    pk = jnp.where(pk < pk[:, 0:1] * mp_ref[...], 0.0, pk)
    val = jnp.log(jnp.maximum(pk, 1e-30)) + g_ref[...]
    gm = jnp.max(val, axis=1, keepdims=True)
    pos = jnp.min(jnp.where(val == gm, rank, jnp.int32(_BIG)), axis=1,
                  keepdims=True)
    o_ref[...] = jnp.sum(jnp.where(rank == pos, i_ref[...], 0), axis=1,
                         keepdims=True)


def _tc_dense(logits, temperatures):
    return pl.pallas_call(
        _dense_body,
        grid=(2, _NB),
        in_specs=[pl.BlockSpec((_B, _WBLK), lambda p, i: (0, i)),
                  pl.BlockSpec((_B, 1), lambda p, i: (0, 0))],
        out_specs=[pl.BlockSpec((_B, _WBLK),
                                lambda p, i: (0, jnp.where(p == 1, i, 0))),
                   pl.BlockSpec((_B, 1), lambda p, i: (0, 0)),
                   pl.BlockSpec((_B, 1), lambda p, i: (0, 0))],
        out_shape=[jax.ShapeDtypeStruct((_B, _V), jnp.float32),
                   jax.ShapeDtypeStruct((_B, 1), jnp.float32),
                   jax.ShapeDtypeStruct((_B, 1), jnp.float32)],
    )(logits, temperatures)


def _tc_sample(vals, idx, temperatures, m, s, top_ks, top_ps, min_ps, g64):
    return pl.pallas_call(
        _sample_body,
        out_shape=jax.ShapeDtypeStruct((_B, 1), jnp.int32),
    )(vals, idx, temperatures, m, s, top_ks, top_ps, min_ps, g64)


def kernel(logits, temperatures, top_ks, top_ps, min_ps):
    vals, idx = _sc_top64(logits)
    vals = vals.reshape(_B, _K)
    idx = idx.reshape(_B, _K)
    logprobs, m, s = _tc_dense(logits, temperatures)
    ids = _tc_sample(vals, idx, temperatures, m, s,
                     top_ks.reshape(_B, 1), top_ps.reshape(_B, 1),
                     min_ps.reshape(_B, 1), _g64())
    return ids.reshape(_B), logprobs
